# baseline (speedup 1.0000x reference)
import functools

import jax
import jax.numpy as jnp
from jax import lax
from jax.experimental import pallas as pl
from jax.experimental.pallas import tpu as pltpu
from jax.experimental.pallas import tpu_sc as plsc

NC = 2
NS = 16
NW = NC * NS
B = 32
PAD = 16
RPAD = 16


def _pre_body(x_ref, wl_ref, bl_ref, wr_ref, br_ref, t_ref):
    x = x_ref[...]
    xl = lax.dot_general(x, wl_ref[...], (((1,), (1,)), ((), ())),
                         preferred_element_type=jnp.float32) + bl_ref[...]
    xr = lax.dot_general(x, wr_ref[...], (((1,), (1,)), ((), ())),
                         preferred_element_type=jnp.float32) + br_ref[...]
    n, d = x.shape
    ones = jnp.ones((n, 1), jnp.float32)
    zc = jnp.zeros((n, PAD - 1), jnp.float32)
    zrows = jnp.zeros((RPAD, d + PAD), jnp.float32)
    zc_r = jnp.zeros((n, PAD), jnp.float32)
    t_ref[...] = jnp.concatenate([
        jnp.concatenate([xl, ones, zc], axis=1), zrows,
        jnp.concatenate([xr, zc_r], axis=1), zrows], axis=0)


def _fin_body(a0_ref, a1_ref, t_ref, att_ref, bias_ref, out_ref):
    n, d = out_ref.shape
    np_ = t_ref.shape[0] // 2
    accsum = a0_ref[...] + a1_ref[...]
    accsum = accsum[:n]
    t = t_ref[...]
    xl = t[:n, :d]
    xr = t[np_:np_ + n, :d]
    s = xl + xr
    lk = jnp.maximum(s, 0.2 * s)
    alpha_self = jnp.sum(lk * att_ref[...], axis=1)
    es = jnp.exp(alpha_self)
    num = accsum[:, :d] + es[:, None] * xl
    den = accsum[:, d] + es + 1e-16
    out_ref[...] = num / den[:, None] + bias_ref[...]


def _edge_kernel_body(t_hbm, iarr_hbm, att_hbm, out_hbm,
                      rows_c, islab, didx, att_v, hsbuf, acc_s,
                      g_sem, s_sem):
    np_ = t_hbm.shape[0] // 2
    dp = t_hbm.shape[1]
    d = dp - PAD
    nchunks = iarr_hbm.shape[0] // NW
    rpt = np_ // NS

    cid = lax.axis_index("c")
    sid = lax.axis_index("s")
    wid = sid * NC + cid

    z16 = jnp.zeros((16,), jnp.float32)

    def zrow(r, carry):
        for k in range(dp // 16):
            rows_c[0, r, pl.ds(k * 16, 16)] = z16
        return carry

    lax.fori_loop(0, B, zrow, 0)
    nfull, rem = rpt // B, rpt % B
    for j in range(nfull):
        pltpu.sync_copy(rows_c.at[0, pl.ds(0, B)],
                        acc_s.at[pl.ds(sid * rpt + j * B, B)])
    if rem:
        pltpu.sync_copy(rows_c.at[0, pl.ds(0, rem)],
                        acc_s.at[pl.ds(sid * rpt + nfull * B, rem)])
    plsc.subcore_barrier()

    pltpu.sync_copy(att_hbm, att_v.at[pl.ds(0, d)])
    slab_rows = islab.shape[0]
    first_fill = min(nchunks, slab_rows)
    pltpu.sync_copy(iarr_hbm.at[pl.ds(wid * nchunks, first_fill)],
                    islab.at[pl.ds(0, first_fill)])

    def srow(c):
        if nchunks <= slab_rows:
            return c
        return lax.select(c < slab_rows, c, c - slab_rows)

    def issue_gather(g, p):
        pltpu.async_copy(t_hbm.at[islab.at[srow(g)]], rows_c.at[p],
                         g_sem.at[p])

    def wait_gather(g, p):
        pltpu.make_async_copy(t_hbm.at[islab.at[srow(g)]], rows_c.at[p],
                              g_sem.at[p]).wait()

    def issue_scatter(g, p):
        pltpu.async_copy(rows_c.at[p, pl.ds(0, B)], acc_s.at[didx.at[p]],
                         s_sem.at[p], add=True)

    def wait_scatter(g, p):
        pltpu.make_async_copy(rows_c.at[p, pl.ds(0, B)],
                              acc_s.at[didx.at[p]], s_sem.at[p]).wait()

    def compute(g, p, att_vs):
        for k in range(B // 16):
            v = islab[srow(g), pl.ds(B + k * 16, 16)]
            didx[p, pl.ds(k * 16, 16)] = v - np_
        iota16 = lax.iota(jnp.int32, 16)
        for t in range(B // 16):
            for j in range(16):
                r = t * 16 + j
                acc = None
                for k in range(d // 16):
                    xlk = rows_c[p, r, pl.ds(k * 16, 16)]
                    xrk = rows_c[p, B + r, pl.ds(k * 16, 16)]
                    s = xlk + xrk
                    lk = jnp.maximum(s, 0.2 * s)
                    term = att_vs[k] * lk
                    acc = term if acc is None else acc + term
                hsbuf[j, pl.ds(0, 16)] = acc
            alpha = jnp.zeros((16,), jnp.float32)
            for c in range(16):
                ccol = jnp.full((16,), c, jnp.int32)
                alpha = alpha + plsc.load_gather(hsbuf, [iota16, ccol])
            expa = jnp.exp(alpha)
            for j in range(16):
                w = expa[j]
                r = t * 16 + j
                for k2 in range(dp // 16):
                    rows_c[p, r, pl.ds(k2 * 16, 16)] = (
                        rows_c[p, r, pl.ds(k2 * 16, 16)] * w)

    nbuf = rows_c.shape[0]
    refill_iter = slab_rows - 5

    issue_gather(0, 0)
    issue_gather(1, 1)

    def body(g, att_vs):
        p = g % nbuf
        wait_gather(g, p)

        if nchunks > slab_rows:
            @pl.when(g == refill_iter)
            def _refill():
                cnt = nchunks - slab_rows
                pltpu.sync_copy(
                    iarr_hbm.at[pl.ds(wid * nchunks + slab_rows, cnt)],
                    islab.at[pl.ds(0, cnt)])

        @pl.when(g + 2 < nchunks)
        def _prefetch():
            b = (g + 2) % nbuf

            @pl.when(g >= 1)
            def _drain():
                wait_scatter(g - 1, b)

            issue_gather(g + 2, b)

        compute(g, p, att_vs)
        issue_scatter(g, p)
        return att_vs

    att_vs0 = tuple(att_v[pl.ds(k * 16, 16)] for k in range(d // 16))
    lax.fori_loop(0, nchunks, body, att_vs0)
    for t in range(min(3, nchunks)):
        c = nchunks - 3 + t
        if c >= 0:
            wait_scatter(c, c % nbuf)

    plsc.subcore_barrier()
    pltpu.sync_copy(acc_s.at[pl.ds(sid * rpt, rpt)],
                    out_hbm.at[cid, pl.ds(sid * rpt, rpt)])


def kernel(x, edge_index, W_l, b_l, W_r, b_r, att, bias):
    n, d = x.shape
    e = edge_index.shape[1]
    dp = d + PAD
    np_ = n + RPAD

    t_tab = pl.pallas_call(
        _pre_body,
        out_shape=jax.ShapeDtypeStruct((2 * np_, dp), jnp.float32),
    )(x, W_l, b_l, W_r, b_r)

    nchunks = -(-e // (NW * B))
    e2 = NW * nchunks * B
    kpad = e2 - e
    src = jnp.concatenate([edge_index[0], jnp.zeros((kpad,), jnp.int32)])
    dst = jnp.concatenate([edge_index[1],
                           jnp.full((kpad,), n, jnp.int32)])
    iarr = jnp.concatenate([src.reshape(NW * nchunks, B),
                            dst.reshape(NW * nchunks, B) + np_], axis=1)

    mesh = plsc.VectorSubcoreMesh(core_axis_name="c", subcore_axis_name="s")
    edge_fn = functools.partial(
        pl.kernel,
        out_type=jax.ShapeDtypeStruct((NC, np_, dp), jnp.float32),
        mesh=mesh,
        scratch_types=[
            pltpu.VMEM((3, 2 * B, dp), jnp.float32),
            pltpu.VMEM((min(nchunks, 160), 2 * B), jnp.int32),
            pltpu.VMEM((3, B), jnp.int32),
            pltpu.VMEM((d + 16,), jnp.float32),
            pltpu.VMEM((16, 17), jnp.float32),
            pltpu.VMEM_SHARED((np_, dp), jnp.float32),
            pltpu.SemaphoreType.DMA((3,)),
            pltpu.SemaphoreType.DMA((3,)),
        ],
        compiler_params=pltpu.CompilerParams(use_tc_tiling_on_sc=False,
                                             needs_layout_passes=False),
    )(_edge_kernel_body)
    acc = edge_fn(t_tab, iarr, att)

    out = pl.pallas_call(
        _fin_body,
        out_shape=jax.ShapeDtypeStruct((n, d), jnp.float32),
    )(acc[0], acc[1], t_tab, att, bias)
    return out
